# baseline (speedup 1.0000x reference)
import jax
import jax.numpy as jnp
from jax.experimental import pallas as pl

SHIFT = 13
NBINS = (0x7F800000 >> SHIFT) + 2


def kernel(logit, target):
    e = (1.0 - logit * (2.0 * target - 1.0)).reshape(-1)
    y = target.reshape(-1)
    bits = jax.lax.bitcast_convert_type(e, jnp.uint32)
    pos = e > 0.0
    b = jnp.where(pos, (bits >> SHIFT).astype(jnp.int32), 0)
    ep = jnp.where(pos, e, 0.0)
    ones = jnp.ones_like(e)
    n = jax.ops.segment_sum(ones, b, num_segments=NBINS)
    n1 = jax.ops.segment_sum(y, b, num_segments=NBINS)
    se = jax.ops.segment_sum(ep, b, num_segments=NBINS)
    se1 = jax.ops.segment_sum(ep * y, b, num_segments=NBINS)
    G = jnp.sum(n1)
    n0 = n - n1
    se0 = se - se1
    c0ge = jnp.cumsum(n0[::-1])[::-1]
    c1ge = jnp.cumsum(n1[::-1])[::-1]
    c0gt = c0ge - n0
    w1 = 1.0 / (G + c0gt)
    w0 = (G - c1ge) / ((G + c0gt) * (G + c0ge))
    contrib = se1 * w1 + se0 * w0
    loss = jnp.sum(contrib.at[0].set(0.0))
    return loss


# speedup vs baseline: 14.3428x; 14.3428x over previous
import functools

import jax
import jax.numpy as jnp
from jax import lax
from jax.experimental import pallas as pl
from jax.experimental.pallas import tpu as pltpu
from jax.experimental.pallas import tpu_sc as plsc

N = 16 * 512 * 512
LANE = 16
SHIFT = 14
NBINS = 1 << 17
HR, HC = 256, 512
NC, NS = 2, 16
NTILES = NC * NS
ROW = 128
ROWS_TOTAL = N // ROW
ROWS_PER_TILE = ROWS_TOTAL // NTILES
CHUNK_ROWS = 128
NCHUNKS = ROWS_PER_TILE // CHUNK_ROWS
SLAB = NBINS // NS

_mesh = plsc.VectorSubcoreMesh(
    core_axis_name="c", subcore_axis_name="s", num_cores=NC, num_subcores=NS)


@functools.partial(
    pl.kernel,
    out_type=jax.ShapeDtypeStruct((NC, 4, NBINS), jnp.float32),
    mesh=_mesh,
    scratch_types=[
        pltpu.VMEM((CHUNK_ROWS, ROW), jnp.float32),
        pltpu.VMEM((CHUNK_ROWS, ROW), jnp.float32),
        pltpu.VMEM((CHUNK_ROWS, ROW), jnp.int32),
        pltpu.VMEM((CHUNK_ROWS, ROW), jnp.float32),
        pltpu.VMEM((CHUNK_ROWS, ROW), jnp.float32),
        pltpu.VMEM((1, ROW), jnp.float32),
        pltpu.VMEM((SLAB,), jnp.float32),
        pltpu.VMEM_SHARED((NBINS,), jnp.float32),
        pltpu.VMEM_SHARED((NBINS,), jnp.float32),
        pltpu.VMEM_SHARED((NBINS,), jnp.float32),
        pltpu.VMEM_SHARED((NBINS,), jnp.float32),
    ],
)
def _sc_hist(logit_hbm, target_hbm, out_hbm,
             logit_v, target_v, bins_v, ep_v, epy_v, ones_v, zero_v,
             h_n, h_n1, h_se, h_se1):
    c = lax.axis_index("c")
    s = lax.axis_index("s")
    tid = c * NS + s

    def fill_zero(i, carry):
        zero_v[pl.ds(i * LANE, LANE)] = jnp.zeros((LANE,), jnp.float32)
        return carry
    lax.fori_loop(0, SLAB // LANE, fill_zero, 0)
    for k in range(ROW // LANE):
        ones_v[0, pl.ds(k * LANE, LANE)] = jnp.ones((LANE,), jnp.float32)

    slab = pl.ds(s * SLAB, SLAB)
    for h in (h_n, h_n1, h_se, h_se1):
        pltpu.sync_copy(zero_v, h.at[slab])
    plsc.subcore_barrier()

    def do_chunk(chunk, carry):
        row0 = tid * ROWS_PER_TILE + chunk * CHUNK_ROWS
        pltpu.sync_copy(logit_hbm.at[pl.ds(row0, CHUNK_ROWS)], logit_v)
        pltpu.sync_copy(target_hbm.at[pl.ds(row0, CHUNK_ROWS)], target_v)

        def do_row(j, inner):
            for k in range(ROW // LANE):
                sl = pl.ds(k * LANE, LANE)
                l = logit_v[j, sl]
                t = target_v[j, sl]
                e = 1.0 - l * (t + t - 1.0)
                pos = e > 0.0
                bits = lax.bitcast_convert_type(e, jnp.int32)
                b = jnp.where(pos, lax.shift_right_logical(bits, SHIFT), 0)
                ep = jnp.where(pos, e, 0.0)
                bins_v[j, sl] = b
                ep_v[j, sl] = ep
                epy_v[j, sl] = ep * t
            idx = bins_v.at[j]
            pltpu.sync_copy(ones_v.at[0], h_n.at[idx], add=True)
            pltpu.sync_copy(target_v.at[j], h_n1.at[idx], add=True)
            pltpu.sync_copy(ep_v.at[j], h_se.at[idx], add=True)
            pltpu.sync_copy(epy_v.at[j], h_se1.at[idx], add=True)
            return inner
        lax.fori_loop(0, CHUNK_ROWS, do_row, 0)
        return carry
    lax.fori_loop(0, NCHUNKS, do_chunk, 0)

    plsc.subcore_barrier()
    for a, h in enumerate((h_n, h_n1, h_se, h_se1)):
        pltpu.sync_copy(h.at[slab], out_hbm.at[c, a, slab])


def _finish_body(h_ref, o_ref):
    h = h_ref[...]
    x = h[0] + h[1]
    n, n1, se, se1 = x[0], x[1], x[2], x[3]
    n0 = n - n1
    se0 = se - se1

    row = lax.broadcasted_iota(jnp.int32, (HR, HC), 0)
    col = lax.broadcasted_iota(jnp.int32, (HR, HC), 1)
    ci = lax.broadcasted_iota(jnp.int32, (HC, HC), 0)
    cj = lax.broadcasted_iota(jnp.int32, (HC, HC), 1)
    upper = (ci <= cj).astype(jnp.float32)
    ri = lax.broadcasted_iota(jnp.int32, (HR, HR), 0)
    rj = lax.broadcasted_iota(jnp.int32, (HR, HR), 1)
    strict = (ri > rj).astype(jnp.float32)

    def suffix_incl(xx):
        pre = lax.dot(xx, upper, precision=lax.Precision.HIGHEST)
        rowsum = jnp.sum(xx, axis=1, keepdims=True)
        off = lax.dot(strict, rowsum, precision=lax.Precision.HIGHEST)
        total = jnp.sum(xx)
        return total - (pre + off) + xx

    c0ge = suffix_incl(n0)
    c1ge = suffix_incl(n1)
    c0gt = c0ge - n0
    G = jnp.sum(n1)

    w1 = 1.0 / (G + c0gt)
    w0 = (G - c1ge) / ((G + c0gt) * (G + c0ge))
    contrib = se1 * w1 + se0 * w0
    first = (row == 0) & (col == 0)
    loss_main = jnp.sum(jnp.where(first, 0.0, contrib))

    binidx = row * HC + col
    m = jnp.max(jnp.where((n > 0.0) & ~first, binidx, 0))
    top_e = lax.bitcast_convert_type((m << SHIFT) + (1 << (SHIFT - 1)),
                                     jnp.float32)
    loss0 = jnp.where(m > 0, top_e, 0.0)

    loss = jnp.where(G > 0.0, loss_main, loss0)
    o_ref[...] = jnp.broadcast_to(loss, (1, 1))


_finish = pl.pallas_call(
    _finish_body,
    out_shape=jax.ShapeDtypeStruct((1, 1), jnp.float32),
)


def kernel(logit, target):
    lf = logit.reshape(ROWS_TOTAL, ROW)
    tf = target.reshape(ROWS_TOTAL, ROW)
    hists = _sc_hist(lf, tf)
    loss = _finish(hists.reshape(NC, 4, HR, HC))
    return loss[0, 0]


# speedup vs baseline: 26.9026x; 1.8757x over previous
import functools

import jax
import jax.numpy as jnp
from jax import lax
from jax.experimental import pallas as pl
from jax.experimental.pallas import tpu as pltpu
from jax.experimental.pallas import tpu_sc as plsc

N = 16 * 512 * 512
LANE = 16
SHIFT = 14
NBINS = 1 << 17
HR, HC = 256, 512
NC, NS = 2, 16
NTILES = NC * NS
ROW = 128
ROWS_TOTAL = N // ROW
ROWS_PER_TILE = ROWS_TOTAL // NTILES
CHUNK_ROWS = 128
NCHUNKS = ROWS_PER_TILE // CHUNK_ROWS
GR = 8
NGROUPS = CHUNK_ROWS // GR
HWORDS = 4 * NBINS
ZSLAB = 8192
SLABS_PER_TILE = HWORDS // NS // ZSLAB

_mesh = plsc.VectorSubcoreMesh(
    core_axis_name="c", subcore_axis_name="s", num_cores=NC, num_subcores=NS)


@functools.partial(
    pl.kernel,
    out_type=jax.ShapeDtypeStruct((NC, HWORDS), jnp.float32),
    mesh=_mesh,
    scratch_types=[
        pltpu.VMEM((CHUNK_ROWS, ROW), jnp.float32),
        pltpu.VMEM((CHUNK_ROWS, ROW), jnp.float32),
        pltpu.VMEM((CHUNK_ROWS, ROW), jnp.int32),
        pltpu.VMEM((CHUNK_ROWS, ROW), jnp.int32),
        pltpu.VMEM((CHUNK_ROWS, ROW), jnp.float32),
        pltpu.VMEM((1, ROW), jnp.float32),
        pltpu.VMEM((ZSLAB,), jnp.float32),
        pltpu.VMEM_SHARED((HWORDS,), jnp.float32),
        pltpu.SemaphoreType.DMA,
    ],
)
def _sc_hist(logit_hbm, target_hbm, out_hbm,
             logit_v, target_v, idxc_v, idxs_v, ep_v, ones_v, zero_v,
             hist, sem):
    c = lax.axis_index("c")
    s = lax.axis_index("s")
    tid = c * NS + s

    def fill_zero(i, carry):
        zero_v[pl.ds(i * LANE, LANE)] = jnp.zeros((LANE,), jnp.float32)
        return carry
    lax.fori_loop(0, ZSLAB // LANE, fill_zero, 0)
    for k in range(ROW // LANE):
        ones_v[0, pl.ds(k * LANE, LANE)] = jnp.ones((LANE,), jnp.float32)

    for q in range(SLABS_PER_TILE):
        off = (s * SLABS_PER_TILE + q) * ZSLAB
        pltpu.sync_copy(zero_v, hist.at[pl.ds(off, ZSLAB)])
    plsc.subcore_barrier()

    def compute_group(j0):
        for r in range(GR):
            j = j0 + r
            for k in range(ROW // LANE):
                sl = pl.ds(k * LANE, LANE)
                l = logit_v[j, sl]
                t = target_v[j, sl]
                e = 1.0 - l * (t + t - 1.0)
                pos = e > 0.0
                bits = lax.bitcast_convert_type(e, jnp.int32)
                b = jnp.where(pos, lax.shift_right_logical(bits, SHIFT), 0)
                ic = t.astype(jnp.int32) * NBINS + b
                idxc_v[j, sl] = ic
                idxs_v[j, sl] = ic + 2 * NBINS
                ep_v[j, sl] = jnp.where(pos, e, 0.0)

    def do_chunk(chunk, carry):
        row0 = tid * ROWS_PER_TILE + chunk * CHUNK_ROWS
        pltpu.sync_copy(logit_hbm.at[pl.ds(row0, CHUNK_ROWS)], logit_v)
        pltpu.sync_copy(target_hbm.at[pl.ds(row0, CHUNK_ROWS)], target_v)
        compute_group(0)

        def do_group(g, inner):
            j0 = g * GR
            descs = []
            for r in range(GR):
                j = j0 + r
                descs.append(pltpu.async_copy(
                    ones_v.at[0], hist.at[idxc_v.at[j]], sem, add=True))
                descs.append(pltpu.async_copy(
                    ep_v.at[j], hist.at[idxs_v.at[j]], sem, add=True))

            @pl.when(g + 1 < NGROUPS)
            def _():
                compute_group((g + 1) * GR)

            for d in descs:
                d.wait()
            return inner
        lax.fori_loop(0, NGROUPS, do_group, 0)
        return carry
    lax.fori_loop(0, NCHUNKS, do_chunk, 0)

    plsc.subcore_barrier()
    for q in range(SLABS_PER_TILE):
        off = (s * SLABS_PER_TILE + q) * ZSLAB
        sl = pl.ds(off, ZSLAB)
        pltpu.sync_copy(hist.at[sl], out_hbm.at[c, sl])


def _finish_body(h_ref, o_ref):
    h = h_ref[...]
    x = h[0] + h[1]
    n0, n1, se0, se1 = x[0], x[1], x[2], x[3]
    n = n0 + n1

    row = lax.broadcasted_iota(jnp.int32, (HR, HC), 0)
    col = lax.broadcasted_iota(jnp.int32, (HR, HC), 1)
    ci = lax.broadcasted_iota(jnp.int32, (HC, HC), 0)
    cj = lax.broadcasted_iota(jnp.int32, (HC, HC), 1)
    upper = (ci <= cj).astype(jnp.float32)
    ri = lax.broadcasted_iota(jnp.int32, (HR, HR), 0)
    rj = lax.broadcasted_iota(jnp.int32, (HR, HR), 1)
    strict = (ri > rj).astype(jnp.float32)

    def suffix_incl(xx):
        pre = lax.dot(xx, upper, precision=lax.Precision.HIGHEST)
        rowsum = jnp.sum(xx, axis=1, keepdims=True)
        off = lax.dot(strict, rowsum, precision=lax.Precision.HIGHEST)
        total = jnp.sum(xx)
        return total - (pre + off) + xx

    c0ge = suffix_incl(n0)
    c1ge = suffix_incl(n1)
    c0gt = c0ge - n0
    G = jnp.sum(n1)

    w1 = 1.0 / (G + c0gt)
    w0 = (G - c1ge) / ((G + c0gt) * (G + c0ge))
    contrib = se1 * w1 + se0 * w0
    first = (row == 0) & (col == 0)
    loss_main = jnp.sum(jnp.where(first, 0.0, contrib))

    binidx = row * HC + col
    m = jnp.max(jnp.where((n > 0.0) & ~first, binidx, 0))
    top_e = lax.bitcast_convert_type((m << SHIFT) + (1 << (SHIFT - 1)),
                                     jnp.float32)
    loss0 = jnp.where(m > 0, top_e, 0.0)

    loss = jnp.where(G > 0.0, loss_main, loss0)
    o_ref[...] = jnp.broadcast_to(loss, (1, 1))


_finish = pl.pallas_call(
    _finish_body,
    out_shape=jax.ShapeDtypeStruct((1, 1), jnp.float32),
)


def kernel(logit, target):
    lf = logit.reshape(ROWS_TOTAL, ROW)
    tf = target.reshape(ROWS_TOTAL, ROW)
    hists = _sc_hist(lf, tf)
    loss = _finish(hists.reshape(NC, 4, HR, HC))
    return loss[0, 0]


# speedup vs baseline: 41.1954x; 1.5313x over previous
import functools

import jax
import jax.numpy as jnp
from jax import lax
from jax.experimental import pallas as pl
from jax.experimental.pallas import tpu as pltpu
from jax.experimental.pallas import tpu_sc as plsc

N = 16 * 512 * 512
LANE = 16
SHIFT = 13
NBINS = 1 << 18
HR, HC = 512, 512
NC, NS = 2, 16
NTILES = NC * NS
ROW = 128
ROWS_TOTAL = N // ROW
ROWS_PER_TILE = ROWS_TOTAL // NTILES
CHUNK_ROWS = 128
NCHUNKS = ROWS_PER_TILE // CHUNK_ROWS
GR = 8
NGROUPS = CHUNK_ROWS // GR
HWORDS = 2 * NBINS
ZSLAB = 8192
SLABS_PER_TILE = HWORDS // NS // ZSLAB

_mesh = plsc.VectorSubcoreMesh(
    core_axis_name="c", subcore_axis_name="s", num_cores=NC, num_subcores=NS)


@functools.partial(
    pl.kernel,
    out_type=jax.ShapeDtypeStruct((NC, HWORDS + 128), jnp.float32),
    mesh=_mesh,
    scratch_types=[
        pltpu.VMEM((CHUNK_ROWS, ROW), jnp.float32),
        pltpu.VMEM((CHUNK_ROWS, ROW), jnp.float32),
        pltpu.VMEM((CHUNK_ROWS, ROW), jnp.int32),
        pltpu.VMEM((CHUNK_ROWS, ROW), jnp.float32),
        pltpu.VMEM((1, LANE), jnp.float32),
        pltpu.VMEM((1, LANE), jnp.int32),
        pltpu.VMEM((ZSLAB,), jnp.float32),
        pltpu.VMEM_SHARED((HWORDS + 128,), jnp.float32),
        pltpu.SemaphoreType.DMA,
    ],
)
def _sc_hist(logit_hbm, target_hbm, out_hbm,
             logit_v, target_v, idx_v, ep_v, gacc_v, gidx_v, zero_v,
             hist, sem):
    c = lax.axis_index("c")
    s = lax.axis_index("s")
    tid = c * NS + s

    def fill_zero(i, carry):
        zero_v[pl.ds(i * LANE, LANE)] = jnp.zeros((LANE,), jnp.float32)
        return carry
    lax.fori_loop(0, ZSLAB // LANE, fill_zero, 0)
    gacc_v[0, :] = jnp.zeros((LANE,), jnp.float32)
    gidx_v[0, :] = HWORDS + lax.iota(jnp.int32, LANE)

    for q in range(SLABS_PER_TILE):
        off = (s * SLABS_PER_TILE + q) * ZSLAB
        pltpu.sync_copy(zero_v, hist.at[pl.ds(off, ZSLAB)])

    @pl.when(s == 0)
    def _():
        pltpu.sync_copy(zero_v.at[pl.ds(0, 128)], hist.at[pl.ds(HWORDS, 128)])
    plsc.subcore_barrier()

    def compute_group(j0):
        tsum = jnp.zeros((LANE,), jnp.float32)
        for r in range(GR):
            j = j0 + r
            for k in range(ROW // LANE):
                sl = pl.ds(k * LANE, LANE)
                l = logit_v[j, sl]
                t = target_v[j, sl]
                e = 1.0 - l * (t + t - 1.0)
                pos = e > 0.0
                bits = lax.bitcast_convert_type(e, jnp.int32)
                b = jnp.where(pos, lax.shift_right_logical(bits, SHIFT), 0)
                idx_v[j, sl] = t.astype(jnp.int32) * NBINS + b
                ep_v[j, sl] = jnp.where(pos, e, 0.0)
                tsum = tsum + t
        gacc_v[0, :] = gacc_v[0, :] + tsum

    def do_chunk(chunk, carry):
        row0 = tid * ROWS_PER_TILE + chunk * CHUNK_ROWS
        pltpu.sync_copy(logit_hbm.at[pl.ds(row0, CHUNK_ROWS)], logit_v)
        pltpu.sync_copy(target_hbm.at[pl.ds(row0, CHUNK_ROWS)], target_v)
        compute_group(0)

        def do_group(g, inner):
            j0 = g * GR
            descs = []
            for r in range(GR):
                j = j0 + r
                descs.append(pltpu.async_copy(
                    ep_v.at[j], hist.at[idx_v.at[j]], sem, add=True))

            @pl.when(g + 1 < NGROUPS)
            def _():
                compute_group((g + 1) * GR)

            for d in descs:
                d.wait()
            return inner
        lax.fori_loop(0, NGROUPS, do_group, 0)
        return carry
    lax.fori_loop(0, NCHUNKS, do_chunk, 0)

    pltpu.sync_copy(gacc_v.at[0], hist.at[gidx_v.at[0]], add=True)

    plsc.subcore_barrier()
    for q in range(SLABS_PER_TILE):
        off = (s * SLABS_PER_TILE + q) * ZSLAB
        sl = pl.ds(off, ZSLAB)
        pltpu.sync_copy(hist.at[sl], out_hbm.at[c, sl])

    @pl.when(s == 0)
    def _():
        tail = pl.ds(HWORDS, 128)
        pltpu.sync_copy(hist.at[tail], out_hbm.at[c, tail])


def _finish_body(se_ref, g_ref, o_ref):
    se = se_ref[...]
    x = se[0] + se[1]
    se0, se1 = x[0], x[1]
    G = jnp.sum(g_ref[...])

    row = lax.broadcasted_iota(jnp.int32, (HR, HC), 0)
    col = lax.broadcasted_iota(jnp.int32, (HR, HC), 1)
    binidx = row * HC + col
    vbits = jnp.minimum((binidx << SHIFT) + (1 << (SHIFT - 1)), 0x7F7FFFFF)
    vbar = lax.bitcast_convert_type(vbits, jnp.float32)
    vbar = jnp.maximum(vbar, 1.2e-38)
    n0 = se0 / vbar
    n1 = se1 / vbar
    first = (row == 0) & (col == 0)
    n0 = jnp.where(first, 0.0, n0)
    n1 = jnp.where(first, 0.0, n1)

    ci = lax.broadcasted_iota(jnp.int32, (HC, HC), 0)
    cj = lax.broadcasted_iota(jnp.int32, (HC, HC), 1)
    upper = (ci <= cj).astype(jnp.float32)
    ri = lax.broadcasted_iota(jnp.int32, (HR, HR), 0)
    rj = lax.broadcasted_iota(jnp.int32, (HR, HR), 1)
    strict = (ri > rj).astype(jnp.float32)

    def suffix_incl(xx):
        pre = lax.dot(xx, upper, precision=lax.Precision.HIGHEST)
        rowsum = jnp.sum(xx, axis=1, keepdims=True)
        off = lax.dot(strict, rowsum, precision=lax.Precision.HIGHEST)
        total = jnp.sum(xx)
        return total - (pre + off) + xx

    c0ge = suffix_incl(n0)
    c1ge = suffix_incl(n1)
    c0gt = c0ge - n0

    w1 = 1.0 / (G + c0gt)
    w0 = jnp.maximum(G - c1ge, 0.0) / ((G + c0gt) * (G + c0ge))
    contrib = se1 * w1 + se0 * w0
    loss_main = jnp.sum(jnp.where(first, 0.0, contrib))

    m = jnp.max(jnp.where(((se0 + se1) > 0.0) & ~first, binidx, 0))
    top_e = lax.bitcast_convert_type((m << SHIFT) + (1 << (SHIFT - 1)),
                                     jnp.float32)
    loss0 = jnp.where(m > 0, top_e, 0.0)

    loss = jnp.where(G > 0.0, loss_main, loss0)
    o_ref[...] = jnp.broadcast_to(loss, (1, 1))


_finish = pl.pallas_call(
    _finish_body,
    out_shape=jax.ShapeDtypeStruct((1, 1), jnp.float32),
)


def kernel(logit, target):
    lf = logit.reshape(ROWS_TOTAL, ROW)
    tf = target.reshape(ROWS_TOTAL, ROW)
    hists = _sc_hist(lf, tf)
    se = hists[:, :HWORDS].reshape(NC, 2, HR, HC)
    gtail = hists[:, HWORDS:]
    loss = _finish(se, gtail)
    return loss[0, 0]


# speedup vs baseline: 41.2157x; 1.0005x over previous
import functools

import jax
import jax.numpy as jnp
from jax import lax
from jax.experimental import pallas as pl
from jax.experimental.pallas import tpu as pltpu
from jax.experimental.pallas import tpu_sc as plsc

N = 16 * 512 * 512
LANE = 16
SHIFT = 13
NBINS = 1 << 18
HR, HC = 512, 512
NC, NS = 2, 16
NTILES = NC * NS
ROW = 128
ROWS_TOTAL = N // ROW
ROWS_PER_TILE = ROWS_TOTAL // NTILES
CHUNK_ROWS = 128
NCHUNKS = ROWS_PER_TILE // CHUNK_ROWS
GR = 8
NGROUPS = CHUNK_ROWS // GR
HWORDS = 2 * NBINS
ZSLAB = 8192
SLABS_PER_TILE = HWORDS // NS // ZSLAB

_mesh = plsc.VectorSubcoreMesh(
    core_axis_name="c", subcore_axis_name="s", num_cores=NC, num_subcores=NS)


@functools.partial(
    pl.kernel,
    out_type=jax.ShapeDtypeStruct((NC, HWORDS + 128), jnp.float32),
    mesh=_mesh,
    scratch_types=[
        pltpu.VMEM((CHUNK_ROWS, ROW), jnp.float32),
        pltpu.VMEM((CHUNK_ROWS, ROW), jnp.float32),
        pltpu.VMEM((CHUNK_ROWS * ROW,), jnp.int32),
        pltpu.VMEM((CHUNK_ROWS * ROW,), jnp.float32),
        pltpu.VMEM((1, LANE), jnp.float32),
        pltpu.VMEM((1, LANE), jnp.int32),
        pltpu.VMEM((ZSLAB,), jnp.float32),
        pltpu.VMEM_SHARED((HWORDS + 128,), jnp.float32),
        pltpu.SemaphoreType.DMA,
    ],
)
def _sc_hist(logit_hbm, target_hbm, out_hbm,
             logit_v, target_v, idx_v, ep_v, gacc_v, gidx_v, zero_v,
             hist, sem):
    c = lax.axis_index("c")
    s = lax.axis_index("s")
    tid = c * NS + s

    def fill_zero(i, carry):
        zero_v[pl.ds(i * LANE, LANE)] = jnp.zeros((LANE,), jnp.float32)
        return carry
    lax.fori_loop(0, ZSLAB // LANE, fill_zero, 0)
    gacc_v[0, :] = jnp.zeros((LANE,), jnp.float32)
    gidx_v[0, :] = HWORDS + lax.iota(jnp.int32, LANE)

    for q in range(SLABS_PER_TILE):
        off = (s * SLABS_PER_TILE + q) * ZSLAB
        pltpu.sync_copy(zero_v, hist.at[pl.ds(off, ZSLAB)])

    @pl.when(s == 0)
    def _():
        pltpu.sync_copy(zero_v.at[pl.ds(0, 128)], hist.at[pl.ds(HWORDS, 128)])
    plsc.subcore_barrier()

    def compute_group(j0):
        tsum = jnp.zeros((LANE,), jnp.float32)
        for r in range(GR):
            j = j0 + r
            for k in range(ROW // LANE):
                sl = pl.ds(k * LANE, LANE)
                l = logit_v[j, sl]
                t = target_v[j, sl]
                e = 1.0 - l * (t + t - 1.0)
                pos = e > 0.0
                bits = lax.bitcast_convert_type(e, jnp.int32)
                b = jnp.where(pos, lax.shift_right_logical(bits, SHIFT), 0)
                fl = pl.ds(j * ROW + k * LANE, LANE)
                idx_v[fl] = t.astype(jnp.int32) * NBINS + b
                ep_v[fl] = jnp.where(pos, e, 0.0)
                tsum = tsum + t
        gacc_v[0, :] = gacc_v[0, :] + tsum

    def do_chunk(chunk, carry):
        row0 = tid * ROWS_PER_TILE + chunk * CHUNK_ROWS
        pltpu.sync_copy(logit_hbm.at[pl.ds(row0, CHUNK_ROWS)], logit_v)
        pltpu.sync_copy(target_hbm.at[pl.ds(row0, CHUNK_ROWS)], target_v)
        compute_group(0)

        def do_group(g, inner):
            j0 = g * GR
            flat = pl.ds(j0 * ROW, GR * ROW)
            desc = pltpu.async_copy(
                ep_v.at[flat], hist.at[idx_v.at[flat]], sem, add=True)

            @pl.when(g + 1 < NGROUPS)
            def _():
                compute_group((g + 1) * GR)

            desc.wait()
            return inner
        lax.fori_loop(0, NGROUPS, do_group, 0)
        return carry
    lax.fori_loop(0, NCHUNKS, do_chunk, 0)

    pltpu.sync_copy(gacc_v.at[0], hist.at[gidx_v.at[0]], add=True)

    plsc.subcore_barrier()
    for q in range(SLABS_PER_TILE):
        off = (s * SLABS_PER_TILE + q) * ZSLAB
        sl = pl.ds(off, ZSLAB)
        pltpu.sync_copy(hist.at[sl], out_hbm.at[c, sl])

    @pl.when(s == 0)
    def _():
        tail = pl.ds(HWORDS, 128)
        pltpu.sync_copy(hist.at[tail], out_hbm.at[c, tail])


def _finish_body(se_ref, g_ref, o_ref):
    se = se_ref[...]
    x = se[0] + se[1]
    se0, se1 = x[0], x[1]
    G = jnp.sum(g_ref[...])

    row = lax.broadcasted_iota(jnp.int32, (HR, HC), 0)
    col = lax.broadcasted_iota(jnp.int32, (HR, HC), 1)
    binidx = row * HC + col
    vbits = jnp.minimum((binidx << SHIFT) + (1 << (SHIFT - 1)), 0x7F7FFFFF)
    vbar = lax.bitcast_convert_type(vbits, jnp.float32)
    vbar = jnp.maximum(vbar, 1.2e-38)
    n0 = se0 / vbar
    n1 = se1 / vbar
    first = (row == 0) & (col == 0)
    n0 = jnp.where(first, 0.0, n0)
    n1 = jnp.where(first, 0.0, n1)

    ci = lax.broadcasted_iota(jnp.int32, (HC, HC), 0)
    cj = lax.broadcasted_iota(jnp.int32, (HC, HC), 1)
    upper = (ci <= cj).astype(jnp.float32)
    ri = lax.broadcasted_iota(jnp.int32, (HR, HR), 0)
    rj = lax.broadcasted_iota(jnp.int32, (HR, HR), 1)
    strict = (ri > rj).astype(jnp.float32)

    def suffix_incl(xx):
        pre = lax.dot(xx, upper, precision=lax.Precision.HIGHEST)
        rowsum = jnp.sum(xx, axis=1, keepdims=True)
        off = lax.dot(strict, rowsum, precision=lax.Precision.HIGHEST)
        total = jnp.sum(xx)
        return total - (pre + off) + xx

    c0ge = suffix_incl(n0)
    c1ge = suffix_incl(n1)
    c0gt = c0ge - n0

    w1 = 1.0 / (G + c0gt)
    w0 = jnp.maximum(G - c1ge, 0.0) / ((G + c0gt) * (G + c0ge))
    contrib = se1 * w1 + se0 * w0
    loss_main = jnp.sum(jnp.where(first, 0.0, contrib))

    m = jnp.max(jnp.where(((se0 + se1) > 0.0) & ~first, binidx, 0))
    top_e = lax.bitcast_convert_type((m << SHIFT) + (1 << (SHIFT - 1)),
                                     jnp.float32)
    loss0 = jnp.where(m > 0, top_e, 0.0)

    loss = jnp.where(G > 0.0, loss_main, loss0)
    o_ref[...] = jnp.broadcast_to(loss, (1, 1))


_finish = pl.pallas_call(
    _finish_body,
    out_shape=jax.ShapeDtypeStruct((1, 1), jnp.float32),
)


def kernel(logit, target):
    lf = logit.reshape(ROWS_TOTAL, ROW)
    tf = target.reshape(ROWS_TOTAL, ROW)
    hists = _sc_hist(lf, tf)
    se = hists[:, :HWORDS].reshape(NC, 2, HR, HC)
    gtail = hists[:, HWORDS:]
    loss = _finish(se, gtail)
    return loss[0, 0]


# speedup vs baseline: 41.4146x; 1.0048x over previous
import functools

import jax
import jax.numpy as jnp
from jax import lax
from jax.experimental import pallas as pl
from jax.experimental.pallas import tpu as pltpu
from jax.experimental.pallas import tpu_sc as plsc

N = 16 * 512 * 512
LANE = 16
SHIFT = 13
NBINS = 1 << 18
HR, HC = 512, 512
NC, NS = 2, 16
NTILES = NC * NS
ROW = 128
ROWS_TOTAL = N // ROW
ROWS_PER_TILE = ROWS_TOTAL // NTILES
CHUNK_ROWS = 128
NCHUNKS = ROWS_PER_TILE // CHUNK_ROWS
GR = 8
NGROUPS = CHUNK_ROWS // GR
HWORDS = 2 * NBINS
ZSLAB = 8192
SLABS_PER_TILE = HWORDS // NS // ZSLAB

_mesh = plsc.VectorSubcoreMesh(
    core_axis_name="c", subcore_axis_name="s", num_cores=NC, num_subcores=NS)


@functools.partial(
    pl.kernel,
    out_type=jax.ShapeDtypeStruct((NC, HWORDS), jnp.float32),
    mesh=_mesh,
    scratch_types=[
        pltpu.VMEM((CHUNK_ROWS, ROW), jnp.float32),
        pltpu.VMEM((CHUNK_ROWS, ROW), jnp.float32),
        pltpu.VMEM((CHUNK_ROWS * ROW,), jnp.int32),
        pltpu.VMEM((CHUNK_ROWS * ROW,), jnp.float32),
        pltpu.VMEM((1, LANE), jnp.float32),
        pltpu.VMEM((1, LANE), jnp.int32),
        pltpu.VMEM((ZSLAB,), jnp.float32),
        pltpu.VMEM_SHARED((HWORDS,), jnp.float32),
        pltpu.SemaphoreType.DMA,
    ],
)
def _sc_hist(logit_hbm, target_hbm, out_hbm,
             logit_v, target_v, idx_v, ep_v, gacc_v, gidx_v, zero_v,
             hist, sem):
    c = lax.axis_index("c")
    s = lax.axis_index("s")
    tid = c * NS + s

    def fill_zero(i, carry):
        zero_v[pl.ds(i * LANE, LANE)] = jnp.zeros((LANE,), jnp.float32)
        return carry
    lax.fori_loop(0, ZSLAB // LANE, fill_zero, 0)
    gacc_v[0, :] = jnp.zeros((LANE,), jnp.float32)
    gidx_v[0, :] = jnp.zeros((LANE,), jnp.int32)

    for q in range(SLABS_PER_TILE):
        off = (s * SLABS_PER_TILE + q) * ZSLAB
        pltpu.sync_copy(zero_v, hist.at[pl.ds(off, ZSLAB)])

    plsc.subcore_barrier()

    def compute_group(j0):
        tsum = jnp.zeros((LANE,), jnp.float32)
        for r in range(GR):
            j = j0 + r
            for k in range(ROW // LANE):
                sl = pl.ds(k * LANE, LANE)
                l = logit_v[j, sl]
                t = target_v[j, sl]
                e = 1.0 - l * (t + t - 1.0)
                pos = e > 0.0
                bits = lax.bitcast_convert_type(e, jnp.int32)
                b = jnp.where(pos, lax.shift_right_logical(bits, SHIFT), 0)
                fl = pl.ds(j * ROW + k * LANE, LANE)
                idx_v[fl] = t.astype(jnp.int32) * NBINS + b
                ep_v[fl] = jnp.where(pos, e, 0.0)
                tsum = tsum + t
        gacc_v[0, :] = gacc_v[0, :] + tsum

    def do_chunk(chunk, carry):
        row0 = tid * ROWS_PER_TILE + chunk * CHUNK_ROWS
        pltpu.sync_copy(logit_hbm.at[pl.ds(row0, CHUNK_ROWS)], logit_v)
        pltpu.sync_copy(target_hbm.at[pl.ds(row0, CHUNK_ROWS)], target_v)
        compute_group(0)

        def do_group(g, inner):
            j0 = g * GR
            flat = pl.ds(j0 * ROW, GR * ROW)
            desc = pltpu.async_copy(
                ep_v.at[flat], hist.at[idx_v.at[flat]], sem, add=True)

            @pl.when(g + 1 < NGROUPS)
            def _():
                compute_group((g + 1) * GR)

            desc.wait()
            return inner
        lax.fori_loop(0, NGROUPS, do_group, 0)
        return carry
    lax.fori_loop(0, NCHUNKS, do_chunk, 0)

    pltpu.sync_copy(gacc_v.at[0], hist.at[gidx_v.at[0]], add=True)

    plsc.subcore_barrier()
    for q in range(SLABS_PER_TILE):
        off = (s * SLABS_PER_TILE + q) * ZSLAB
        sl = pl.ds(off, ZSLAB)
        pltpu.sync_copy(hist.at[sl], out_hbm.at[c, sl])


def _finish_body(se_ref, o_ref):
    se = se_ref[...]
    x = se[0] + se[1]
    se0, se1 = x[0], x[1]
    G = x[0, 0, 0]

    row = lax.broadcasted_iota(jnp.int32, (HR, HC), 0)
    col = lax.broadcasted_iota(jnp.int32, (HR, HC), 1)
    binidx = row * HC + col
    vbits = jnp.minimum((binidx << SHIFT) + (1 << (SHIFT - 1)), 0x7F7FFFFF)
    vbar = lax.bitcast_convert_type(vbits, jnp.float32)
    vbar = jnp.maximum(vbar, 1.2e-38)
    n0 = se0 / vbar
    n1 = se1 / vbar
    first = (row == 0) & (col == 0)
    n0 = jnp.where(first, 0.0, n0)
    n1 = jnp.where(first, 0.0, n1)

    ci = lax.broadcasted_iota(jnp.int32, (HC, HC), 0)
    cj = lax.broadcasted_iota(jnp.int32, (HC, HC), 1)
    upper = (ci <= cj).astype(jnp.float32)
    ri = lax.broadcasted_iota(jnp.int32, (HR, HR), 0)
    rj = lax.broadcasted_iota(jnp.int32, (HR, HR), 1)
    strict = (ri > rj).astype(jnp.float32)

    def suffix_incl(xx):
        pre = lax.dot(xx, upper, precision=lax.Precision.HIGHEST)
        rowsum = jnp.sum(xx, axis=1, keepdims=True)
        off = lax.dot(strict, rowsum, precision=lax.Precision.HIGHEST)
        total = jnp.sum(xx)
        return total - (pre + off) + xx

    c0ge = suffix_incl(n0)
    c1ge = suffix_incl(n1)
    c0gt = c0ge - n0

    w1 = 1.0 / (G + c0gt)
    w0 = jnp.maximum(G - c1ge, 0.0) / ((G + c0gt) * (G + c0ge))
    contrib = se1 * w1 + se0 * w0
    loss_main = jnp.sum(jnp.where(first, 0.0, contrib))

    m = jnp.max(jnp.where(((se0 + se1) > 0.0) & ~first, binidx, 0))
    top_e = lax.bitcast_convert_type((m << SHIFT) + (1 << (SHIFT - 1)),
                                     jnp.float32)
    loss0 = jnp.where(m > 0, top_e, 0.0)

    loss = jnp.where(G > 0.0, loss_main, loss0)
    o_ref[...] = jnp.broadcast_to(loss, (1, 1))


_finish = pl.pallas_call(
    _finish_body,
    out_shape=jax.ShapeDtypeStruct((1, 1), jnp.float32),
)


def kernel(logit, target):
    lf = logit.reshape(ROWS_TOTAL, ROW)
    tf = target.reshape(ROWS_TOTAL, ROW)
    hists = _sc_hist(lf, tf)
    loss = _finish(hists.reshape(NC, 2, HR, HC))
    return loss[0, 0]


# speedup vs baseline: 41.4299x; 1.0004x over previous
import functools

import jax
import jax.numpy as jnp
from jax import lax
from jax.experimental import pallas as pl
from jax.experimental.pallas import tpu as pltpu
from jax.experimental.pallas import tpu_sc as plsc

N = 16 * 512 * 512
LANE = 16
SHIFT = 13
NBINS = 1 << 18
HR, HC = 512, 512
NC, NS = 2, 16
NTILES = NC * NS
ROW = 128
ROWS_TOTAL = N // ROW
ROWS_PER_TILE = ROWS_TOTAL // NTILES
CHUNK_ROWS = 128
NCHUNKS = ROWS_PER_TILE // CHUNK_ROWS
GR = 8
NGROUPS = CHUNK_ROWS // GR
HWORDS = 2 * NBINS
ZSLAB = 8192
SLABS_PER_TILE = HWORDS // NS // ZSLAB

_mesh = plsc.VectorSubcoreMesh(
    core_axis_name="c", subcore_axis_name="s", num_cores=NC, num_subcores=NS)


@functools.partial(
    pl.kernel,
    out_type=jax.ShapeDtypeStruct((NC, HWORDS), jnp.float32),
    mesh=_mesh,
    scratch_types=[
        pltpu.VMEM((CHUNK_ROWS, ROW), jnp.float32),
        pltpu.VMEM((CHUNK_ROWS, ROW), jnp.float32),
        pltpu.VMEM((CHUNK_ROWS * ROW,), jnp.int32),
        pltpu.VMEM((CHUNK_ROWS * ROW,), jnp.float32),
        pltpu.VMEM((1, LANE), jnp.float32),
        pltpu.VMEM((1, LANE), jnp.int32),
        pltpu.VMEM((ZSLAB,), jnp.float32),
        pltpu.VMEM_SHARED((HWORDS,), jnp.float32),
        pltpu.SemaphoreType.DMA,
    ],
)
def _sc_hist(logit_hbm, target_hbm, out_hbm,
             logit_v, target_v, idx_v, ep_v, gacc_v, gidx_v, zero_v,
             hist, sem):
    c = lax.axis_index("c")
    s = lax.axis_index("s")
    tid = c * NS + s

    def fill_zero(i, carry):
        zero_v[pl.ds(i * LANE, LANE)] = jnp.zeros((LANE,), jnp.float32)
        return carry
    lax.fori_loop(0, ZSLAB // LANE, fill_zero, 0)
    gacc_v[0, :] = jnp.zeros((LANE,), jnp.float32)
    gidx_v[0, :] = jnp.zeros((LANE,), jnp.int32)

    for q in range(SLABS_PER_TILE):
        off = (s * SLABS_PER_TILE + q) * ZSLAB
        pltpu.sync_copy(zero_v, hist.at[pl.ds(off, ZSLAB)])

    plsc.subcore_barrier()

    def compute_group(j0):
        tsum = jnp.zeros((LANE,), jnp.float32)
        for r in range(GR):
            j = j0 + r
            for k in range(ROW // LANE):
                sl = pl.ds(k * LANE, LANE)
                l = logit_v[j, sl]
                t = target_v[j, sl]
                e = 1.0 - l * (t + t - 1.0)
                pos = e > 0.0
                bits = lax.bitcast_convert_type(e, jnp.int32)
                b = jnp.where(pos, lax.shift_right_logical(bits, SHIFT), 0)
                fl = pl.ds(j * ROW + k * LANE, LANE)
                idx_v[fl] = t.astype(jnp.int32) * NBINS + b
                ep_v[fl] = jnp.where(pos, e, 0.0)
                tsum = tsum + t
        gacc_v[0, :] = gacc_v[0, :] + tsum

    def do_chunk(chunk, carry):
        row0 = tid * ROWS_PER_TILE + chunk * CHUNK_ROWS
        pltpu.sync_copy(logit_hbm.at[pl.ds(row0, CHUNK_ROWS)], logit_v)
        pltpu.sync_copy(target_hbm.at[pl.ds(row0, CHUNK_ROWS)], target_v)
        compute_group(0)

        def do_group(g, inner):
            j0 = g * GR
            half = GR * ROW // 2
            fa = pl.ds(j0 * ROW, half)
            fb = pl.ds(j0 * ROW + half, half)
            da = pltpu.async_copy(
                ep_v.at[fa], hist.at[idx_v.at[fa]], sem, add=True)
            db = pltpu.async_copy(
                ep_v.at[fb], hist.at[idx_v.at[fb]], sem, add=True)

            @pl.when(g + 1 < NGROUPS)
            def _():
                compute_group((g + 1) * GR)

            da.wait()
            db.wait()
            return inner
        lax.fori_loop(0, NGROUPS, do_group, 0)
        return carry
    lax.fori_loop(0, NCHUNKS, do_chunk, 0)

    pltpu.sync_copy(gacc_v.at[0], hist.at[gidx_v.at[0]], add=True)

    plsc.subcore_barrier()
    for q in range(SLABS_PER_TILE):
        off = (s * SLABS_PER_TILE + q) * ZSLAB
        sl = pl.ds(off, ZSLAB)
        pltpu.sync_copy(hist.at[sl], out_hbm.at[c, sl])


def _finish_body(se_ref, o_ref):
    se = se_ref[...]
    x = se[0] + se[1]
    se0, se1 = x[0], x[1]
    G = x[0, 0, 0]

    row = lax.broadcasted_iota(jnp.int32, (HR, HC), 0)
    col = lax.broadcasted_iota(jnp.int32, (HR, HC), 1)
    binidx = row * HC + col
    vbits = jnp.minimum((binidx << SHIFT) + (1 << (SHIFT - 1)), 0x7F7FFFFF)
    vbar = lax.bitcast_convert_type(vbits, jnp.float32)
    vbar = jnp.maximum(vbar, 1.2e-38)
    n0 = se0 / vbar
    n1 = se1 / vbar
    first = (row == 0) & (col == 0)
    n0 = jnp.where(first, 0.0, n0)
    n1 = jnp.where(first, 0.0, n1)

    ci = lax.broadcasted_iota(jnp.int32, (HC, HC), 0)
    cj = lax.broadcasted_iota(jnp.int32, (HC, HC), 1)
    upper = (ci <= cj).astype(jnp.float32)
    ri = lax.broadcasted_iota(jnp.int32, (HR, HR), 0)
    rj = lax.broadcasted_iota(jnp.int32, (HR, HR), 1)
    strict = (ri > rj).astype(jnp.float32)

    def suffix_incl(xx):
        pre = lax.dot(xx, upper, precision=lax.Precision.HIGHEST)
        rowsum = jnp.sum(xx, axis=1, keepdims=True)
        off = lax.dot(strict, rowsum, precision=lax.Precision.HIGHEST)
        total = jnp.sum(xx)
        return total - (pre + off) + xx

    c0ge = suffix_incl(n0)
    c1ge = suffix_incl(n1)
    c0gt = c0ge - n0

    w1 = 1.0 / (G + c0gt)
    w0 = jnp.maximum(G - c1ge, 0.0) / ((G + c0gt) * (G + c0ge))
    contrib = se1 * w1 + se0 * w0
    loss_main = jnp.sum(jnp.where(first, 0.0, contrib))

    m = jnp.max(jnp.where(((se0 + se1) > 0.0) & ~first, binidx, 0))
    top_e = lax.bitcast_convert_type((m << SHIFT) + (1 << (SHIFT - 1)),
                                     jnp.float32)
    loss0 = jnp.where(m > 0, top_e, 0.0)

    loss = jnp.where(G > 0.0, loss_main, loss0)
    o_ref[...] = jnp.broadcast_to(loss, (1, 1))


_finish = pl.pallas_call(
    _finish_body,
    out_shape=jax.ShapeDtypeStruct((1, 1), jnp.float32),
)


def kernel(logit, target):
    lf = logit.reshape(ROWS_TOTAL, ROW)
    tf = target.reshape(ROWS_TOTAL, ROW)
    hists = _sc_hist(lf, tf)
    loss = _finish(hists.reshape(NC, 2, HR, HC))
    return loss[0, 0]


# speedup vs baseline: 42.6100x; 1.0285x over previous
import functools

import jax
import jax.numpy as jnp
from jax import lax
from jax.experimental import pallas as pl
from jax.experimental.pallas import tpu as pltpu
from jax.experimental.pallas import tpu_sc as plsc

N = 16 * 512 * 512
LANE = 16
SHIFT = 15
NBINS = 1 << 16
HR, HC = 128, 512
NC, NS = 2, 16
NTILES = NC * NS
ROW = 128
ROWS_TOTAL = N // ROW
ROWS_PER_TILE = ROWS_TOTAL // NTILES
CHUNK_ROWS = 128
NCHUNKS = ROWS_PER_TILE // CHUNK_ROWS
GR = 8
NGROUPS = CHUNK_ROWS // GR
HWORDS = 2 * NBINS
ZSLAB = 8192
SLABS_PER_TILE = HWORDS // NS // ZSLAB

_mesh = plsc.VectorSubcoreMesh(
    core_axis_name="c", subcore_axis_name="s", num_cores=NC, num_subcores=NS)


@functools.partial(
    pl.kernel,
    out_type=jax.ShapeDtypeStruct((NC, HWORDS), jnp.float32),
    mesh=_mesh,
    scratch_types=[
        pltpu.VMEM((CHUNK_ROWS, ROW), jnp.float32),
        pltpu.VMEM((CHUNK_ROWS, ROW), jnp.float32),
        pltpu.VMEM((CHUNK_ROWS * ROW,), jnp.int32),
        pltpu.VMEM((CHUNK_ROWS * ROW,), jnp.float32),
        pltpu.VMEM((1, LANE), jnp.float32),
        pltpu.VMEM((1, LANE), jnp.int32),
        pltpu.VMEM((ZSLAB,), jnp.float32),
        pltpu.VMEM_SHARED((HWORDS,), jnp.float32),
        pltpu.SemaphoreType.DMA,
    ],
)
def _sc_hist(logit_hbm, target_hbm, out_hbm,
             logit_v, target_v, idx_v, ep_v, gacc_v, gidx_v, zero_v,
             hist, sem):
    c = lax.axis_index("c")
    s = lax.axis_index("s")
    tid = c * NS + s

    def fill_zero(i, carry):
        zero_v[pl.ds(i * LANE, LANE)] = jnp.zeros((LANE,), jnp.float32)
        return carry
    lax.fori_loop(0, ZSLAB // LANE, fill_zero, 0)
    gacc_v[0, :] = jnp.zeros((LANE,), jnp.float32)
    gidx_v[0, :] = jnp.zeros((LANE,), jnp.int32)

    for q in range(SLABS_PER_TILE):
        off = (s * SLABS_PER_TILE + q) * ZSLAB
        pltpu.sync_copy(zero_v, hist.at[pl.ds(off, ZSLAB)])

    plsc.subcore_barrier()

    def compute_group(j0):
        tsum = jnp.zeros((LANE,), jnp.float32)
        for r in range(GR):
            j = j0 + r
            for k in range(ROW // LANE):
                sl = pl.ds(k * LANE, LANE)
                l = logit_v[j, sl]
                t = target_v[j, sl]
                e = 1.0 - l * (t + t - 1.0)
                bits = lax.bitcast_convert_type(e, jnp.int32)
                b = lax.shift_right_logical(jnp.maximum(bits, 0), SHIFT)
                fl = pl.ds(j * ROW + k * LANE, LANE)
                idx_v[fl] = lax.shift_left(t.astype(jnp.int32), 16) + b
                ep_v[fl] = jnp.maximum(e, 0.0)
                tsum = tsum + t
        gacc_v[0, :] = gacc_v[0, :] + tsum

    def do_chunk(chunk, carry):
        row0 = tid * ROWS_PER_TILE + chunk * CHUNK_ROWS
        pltpu.sync_copy(logit_hbm.at[pl.ds(row0, CHUNK_ROWS)], logit_v)
        pltpu.sync_copy(target_hbm.at[pl.ds(row0, CHUNK_ROWS)], target_v)
        compute_group(0)

        def do_group(g, inner):
            j0 = g * GR
            half = GR * ROW // 2
            fa = pl.ds(j0 * ROW, half)
            fb = pl.ds(j0 * ROW + half, half)
            da = pltpu.async_copy(
                ep_v.at[fa], hist.at[idx_v.at[fa]], sem, add=True)
            db = pltpu.async_copy(
                ep_v.at[fb], hist.at[idx_v.at[fb]], sem, add=True)

            @pl.when(g + 1 < NGROUPS)
            def _():
                compute_group((g + 1) * GR)

            da.wait()
            db.wait()
            return inner
        lax.fori_loop(0, NGROUPS, do_group, 0)
        return carry
    lax.fori_loop(0, NCHUNKS, do_chunk, 0)

    pltpu.sync_copy(gacc_v.at[0], hist.at[gidx_v.at[0]], add=True)

    plsc.subcore_barrier()
    for q in range(SLABS_PER_TILE):
        off = (s * SLABS_PER_TILE + q) * ZSLAB
        sl = pl.ds(off, ZSLAB)
        pltpu.sync_copy(hist.at[sl], out_hbm.at[c, sl])


def _finish_body(se_ref, o_ref):
    se = se_ref[...]
    x = se[0] + se[1]
    se0, se1 = x[0], x[1]
    G = x[0, 0, 0]

    row = lax.broadcasted_iota(jnp.int32, (HR, HC), 0)
    col = lax.broadcasted_iota(jnp.int32, (HR, HC), 1)
    binidx = row * HC + col
    vbits = jnp.minimum((binidx << SHIFT) + (1 << (SHIFT - 1)), 0x7F7FFFFF)
    vbar = lax.bitcast_convert_type(vbits, jnp.float32)
    vbar = jnp.maximum(vbar, 1.2e-38)
    n0 = se0 / vbar
    n1 = se1 / vbar
    first = (row == 0) & (col == 0)
    n0 = jnp.where(first, 0.0, n0)
    n1 = jnp.where(first, 0.0, n1)

    ci = lax.broadcasted_iota(jnp.int32, (HC, HC), 0)
    cj = lax.broadcasted_iota(jnp.int32, (HC, HC), 1)
    upper = (ci <= cj).astype(jnp.float32)
    ri = lax.broadcasted_iota(jnp.int32, (HR, HR), 0)
    rj = lax.broadcasted_iota(jnp.int32, (HR, HR), 1)
    strict = (ri > rj).astype(jnp.float32)

    def suffix_incl(xx):
        pre = lax.dot(xx, upper, precision=lax.Precision.HIGHEST)
        rowsum = jnp.sum(xx, axis=1, keepdims=True)
        off = lax.dot(strict, rowsum, precision=lax.Precision.HIGHEST)
        total = jnp.sum(xx)
        return total - (pre + off) + xx

    c0ge = suffix_incl(n0)
    c1ge = suffix_incl(n1)
    c0gt = c0ge - n0

    w1 = 1.0 / (G + c0gt)
    w0 = jnp.maximum(G - c1ge, 0.0) / ((G + c0gt) * (G + c0ge))
    contrib = se1 * w1 + se0 * w0
    loss_main = jnp.sum(jnp.where(first, 0.0, contrib))

    m = jnp.max(jnp.where(((se0 + se1) > 0.0) & ~first, binidx, 0))
    top_e = lax.bitcast_convert_type((m << SHIFT) + (1 << (SHIFT - 1)),
                                     jnp.float32)
    loss0 = jnp.where(m > 0, top_e, 0.0)

    loss = jnp.where(G > 0.0, loss_main, loss0)
    o_ref[...] = jnp.broadcast_to(loss, (1, 1))


_finish = pl.pallas_call(
    _finish_body,
    out_shape=jax.ShapeDtypeStruct((1, 1), jnp.float32),
)


def kernel(logit, target):
    lf = logit.reshape(ROWS_TOTAL, ROW)
    tf = target.reshape(ROWS_TOTAL, ROW)
    hists = _sc_hist(lf, tf)
    loss = _finish(hists.reshape(NC, 2, HR, HC))
    return loss[0, 0]
